# Initial kernel scaffold; baseline (speedup 1.0000x reference)
import jax
import jax.numpy as jnp
from jax.experimental import pallas as pl


def kernel(indices, table, discrete_indices, offsets):
    raise NotImplementedError("write your pallas kernel here")


# speedup vs baseline: 3.2493x; 3.2493x over previous
import functools
import math

import jax
import jax.numpy as jnp
from jax import lax
from jax.experimental import pallas as pl
from jax.experimental.pallas import tpu as pltpu
from jax.experimental.pallas import tpu_sc as plsc


def kernel(indices, table, discrete_indices, offsets):
    B, S = indices.shape
    V, D = table.shape
    P = discrete_indices.shape[0]

    info = plsc.get_sparse_core_info()
    NC, NS, L = info.num_cores, info.num_subcores, info.num_lanes
    NW = NC * NS
    N = B * S
    assert N % NW == 0
    NR = N // NW
    C = 512
    assert NR % C == 0
    NCHUNK = NR // C
    assert C % L == 0
    PERIOD = math.lcm(L, S)
    assert NR % PERIOD == 0

    mesh = plsc.VectorSubcoreMesh(core_axis_name="c", subcore_axis_name="s")

    @functools.partial(
        pl.kernel,
        mesh=mesh,
        out_type=jax.ShapeDtypeStruct((N, D), jnp.float32),
        compiler_params=pltpu.CompilerParams(use_tc_tiling_on_sc=False),
        scratch_types=[
            pltpu.VMEM((NR,), jnp.int32),
            pltpu.VMEM((NR,), jnp.int32),
            pltpu.VMEM((PERIOD,), jnp.int32),
            pltpu.VMEM((PERIOD,), jnp.int32),
            pltpu.VMEM((C, D), jnp.float32),
            pltpu.SemaphoreType.DMA,
        ],
    )
    def run(idx_hbm, table_hbm, disc_hbm, offs_hbm, out_hbm,
            idx_v, eidx_v, spat_v, pat_v, rows_v, sem):
        wid = lax.axis_index("c") * NS + lax.axis_index("s")
        base = wid * NR

        pltpu.sync_copy(idx_hbm.at[pl.ds(base, NR)], idx_v)

        lane = lax.iota(jnp.int32, L)

        def mk_s(i, _):
            jb = i * L
            spat_v[pl.ds(jb, L)] = (jnp.full((L,), jb, jnp.int32) + lane) % S
            return 0

        lax.fori_loop(0, PERIOD // L, mk_s, 0)

        pltpu.async_copy(offs_hbm.at[spat_v], pat_v, sem).wait()

        def mk_p(i, _):
            jb = i * L
            idx_v[pl.ds(jb, L)] = (
                idx_v[pl.ds(jb, L)] + pat_v[pl.ds((i % (PERIOD // L)) * L, L)]
            )
            return 0

        lax.fori_loop(0, NR // L, mk_p, 0, unroll=2)

        pltpu.async_copy(disc_hbm.at[idx_v], eidx_v, sem).wait()

        def chunk(c, _):
            pltpu.async_copy(
                table_hbm.at[eidx_v.at[pl.ds(c * C, C)]], rows_v, sem
            ).wait()
            pltpu.sync_copy(rows_v, out_hbm.at[pl.ds(base + c * C, C)])
            return 0

        lax.fori_loop(0, NCHUNK, chunk, 0)

    out = run(indices.reshape(N), table, discrete_indices, offsets)
    return out.reshape(B, S, D)


# speedup vs baseline: 4.1259x; 1.2698x over previous
import functools
import math

import jax
import jax.numpy as jnp
from jax import lax
from jax.experimental import pallas as pl
from jax.experimental.pallas import tpu as pltpu
from jax.experimental.pallas import tpu_sc as plsc


def kernel(indices, table, discrete_indices, offsets):
    B, S = indices.shape
    V, D = table.shape
    P = discrete_indices.shape[0]

    info = plsc.get_sparse_core_info()
    NC, NS, L = info.num_cores, info.num_subcores, info.num_lanes
    NW = NC * NS
    N = B * S
    assert N % NW == 0
    NR = N // NW
    C = 416
    assert NR % (2 * C) == 0
    NCHUNK = NR // C
    assert C % L == 0
    PERIOD = math.lcm(L, S)
    assert NR % PERIOD == 0
    assert D % L == 0

    mesh = plsc.VectorSubcoreMesh(core_axis_name="c", subcore_axis_name="s")

    @functools.partial(
        pl.kernel,
        mesh=mesh,
        out_type=jax.ShapeDtypeStruct((N, D), jnp.float32),
        compiler_params=pltpu.CompilerParams(
            use_tc_tiling_on_sc=False, needs_layout_passes=False
        ),
        scratch_types=[
            pltpu.VMEM((NR,), jnp.int32),
            pltpu.VMEM((128,), jnp.int32),
            pltpu.VMEM((PERIOD,), jnp.int32),
            pltpu.VMEM((P,), jnp.int32),
            pltpu.VMEM((P, D), jnp.float32),
            pltpu.VMEM((2, C, D), jnp.float32),
            pltpu.SemaphoreType.DMA,
            pltpu.SemaphoreType.DMA,
            pltpu.SemaphoreType.DMA,
        ],
    )
    def run(idx_hbm, table_hbm, disc_hbm, offs_hbm, out_hbm,
            idx_v, offs_v, pat_v, disc_v, sub_v, rows_v, gsem, sem0, sem1):
        wid = lax.axis_index("c") * NS + lax.axis_index("s")
        base = wid * NR

        idx_cp = pltpu.async_copy(idx_hbm.at[pl.ds(base, NR)], idx_v, gsem)
        pltpu.sync_copy(disc_hbm, disc_v)
        pltpu.sync_copy(offs_hbm, offs_v.at[pl.ds(0, S)])

        sub_cp = pltpu.async_copy(table_hbm.at[disc_v], sub_v, gsem)

        lane = lax.iota(jnp.int32, L)

        def mk_pat(i, _):
            jb = i * L
            sv = (jnp.full((L,), jb, jnp.int32) + lane) % S
            pat_v[pl.ds(jb, L)] = plsc.load_gather(offs_v, [sv])
            return 0

        lax.fori_loop(0, PERIOD // L, mk_pat, 0)

        idx_cp.wait()
        sub_cp.wait()

        GPC = C // L
        NPAT = PERIOD // L

        def assemble(g, buf):
            jb = g * L
            pk = idx_v[pl.ds(jb, L)] + pat_v[pl.ds((g % NPAT) * L, L)]
            r = (g % GPC) * L
            rbuf = rows_v.at[buf]
            for w in range(D):
                wv = jnp.full((L,), w, jnp.int32)
                vals = plsc.load_gather(sub_v, [pk, wv])
                plsc.store_scatter(rbuf, [lane + r, wv], vals)
            return None

        def two_chunks(h, _):
            for par in range(2):
                c = h * 2 + par
                sem = sem0 if par == 0 else sem1

                @pl.when(h > 0)
                def _wait():
                    pltpu.make_async_copy(
                        rows_v.at[par], out_hbm.at[pl.ds(0, C)], sem
                    ).wait()

                def inner(g, _):
                    assemble(c * GPC + g, par)
                    return 0

                lax.fori_loop(0, GPC, inner, 0)
                pltpu.make_async_copy(
                    rows_v.at[par],
                    out_hbm.at[pl.ds(base + c * C, C)],
                    sem,
                ).start()
            return 0

        lax.fori_loop(0, NCHUNK // 2, two_chunks, 0)

        for sem in (sem0, sem1):
            pltpu.make_async_copy(
                rows_v.at[0], out_hbm.at[pl.ds(0, C)], sem
            ).wait()

    out = run(indices.reshape(N), table, discrete_indices, offsets)
    return out.reshape(B, S, D)


# speedup vs baseline: 13.4994x; 3.2719x over previous
import functools
import math

import jax
import jax.numpy as jnp
from jax import lax
from jax.experimental import pallas as pl
from jax.experimental.pallas import tpu as pltpu
from jax.experimental.pallas import tpu_sc as plsc


def kernel(indices, table, discrete_indices, offsets):
    B, S = indices.shape
    V, D = table.shape
    P = discrete_indices.shape[0]

    info = plsc.get_sparse_core_info()
    NC, NS, L = info.num_cores, info.num_subcores, info.num_lanes
    NW = NC * NS
    N = B * S
    assert N % NW == 0
    NR = N // NW
    C = 416
    assert NR % (2 * C) == 0
    NCHUNK = NR // C
    assert C % L == 0
    PERIOD = math.lcm(L, S)
    assert NR % PERIOD == 0
    assert D % L == 0

    mesh = plsc.VectorSubcoreMesh(core_axis_name="c", subcore_axis_name="s")

    @functools.partial(
        pl.kernel,
        mesh=mesh,
        out_type=jax.ShapeDtypeStruct((N, D), jnp.float32),
        compiler_params=pltpu.CompilerParams(
            use_tc_tiling_on_sc=False, needs_layout_passes=False
        ),
        scratch_types=[
            pltpu.VMEM((NR,), jnp.int32),
            pltpu.VMEM((128,), jnp.int32),
            pltpu.VMEM((PERIOD,), jnp.int32),
            pltpu.VMEM((P,), jnp.int32),
            pltpu.VMEM_SHARED((P, D), jnp.float32),
            pltpu.VMEM((P, D), jnp.float32),
            pltpu.VMEM((2, C, D), jnp.float32),
            pltpu.SemaphoreType.DMA,
            pltpu.SemaphoreType.DMA,
            pltpu.SemaphoreType.DMA,
            pltpu.SemaphoreType.DMA,
        ],
    )
    def run(idx_hbm, table_hbm, disc_hbm, offs_hbm, out_hbm,
            idx_v, offs_v, pat_v, disc_v, sub_v, stage_v, rows_v,
            gsem, asem, sem0, sem1):
        wid = lax.axis_index("c") * NS + lax.axis_index("s")
        base = wid * NR

        idx_cp = pltpu.async_copy(idx_hbm.at[pl.ds(base, NR)], idx_v, gsem)
        pltpu.sync_copy(disc_hbm, disc_v)
        pltpu.sync_copy(offs_hbm, offs_v.at[pl.ds(0, S)])

        @pl.when(lax.axis_index("s") == 0)
        def _build_sub():
            pltpu.async_copy(table_hbm.at[disc_v], stage_v, gsem).wait()
            pltpu.sync_copy(stage_v, sub_v)

        lane = lax.iota(jnp.int32, L)

        def mk_pat(i, _):
            jb = i * L
            sv = (jnp.full((L,), jb, jnp.int32) + lane) % S
            pat_v[pl.ds(jb, L)] = plsc.load_gather(offs_v, [sv])
            return 0

        lax.fori_loop(0, PERIOD // L, mk_pat, 0)

        idx_cp.wait()

        NPAT = PERIOD // L

        def mk_packed(i, _):
            jb = i * L
            idx_v[pl.ds(jb, L)] = (
                idx_v[pl.ds(jb, L)] + pat_v[pl.ds((i % NPAT) * L, L)]
            )
            return 0

        lax.fori_loop(0, NR // L, mk_packed, 0, unroll=4)

        plsc.subcore_barrier()

        def two_chunks(h, _):
            for par in range(2):
                c = h * 2 + par
                sem = sem0 if par == 0 else sem1

                @pl.when(h > 0)
                def _wait():
                    pltpu.make_async_copy(
                        rows_v.at[par], out_hbm.at[pl.ds(0, C)], sem
                    ).wait()

                pltpu.async_copy(
                    sub_v.at[idx_v.at[pl.ds(c * C, C)]], rows_v.at[par], asem
                ).wait()
                pltpu.make_async_copy(
                    rows_v.at[par],
                    out_hbm.at[pl.ds(base + c * C, C)],
                    sem,
                ).start()
            return 0

        lax.fori_loop(0, NCHUNK // 2, two_chunks, 0)

        for sem in (sem0, sem1):
            pltpu.make_async_copy(
                rows_v.at[0], out_hbm.at[pl.ds(0, C)], sem
            ).wait()

    out = run(indices.reshape(N), table, discrete_indices, offsets)
    return out.reshape(B, S, D)
